# baseline (speedup 1.0000x reference)
import jax
import jax.numpy as jnp
from jax import lax
from jax.experimental import pallas as pl
from jax.experimental.pallas import tpu as pltpu
from jax.experimental.pallas import tpu_sc as plsc

B = 16384
F = 26
V1 = 100001
D = 16
NC = 2
NS = 16
NW = NC * NS
CHUNK = 128
ROWS_PW = B // NW
CPW = ROWS_PW // CHUNK


def _fm_body(tab_hbm, idx_hbm, idxf_hbm, w_hbm, out_hbm,
             idx_v, rows_v, idxf_v, out_v, w_v, sem0, sem1):
    wid = lax.axis_index("s") * NC + lax.axis_index("c")
    base = wid * ROWS_PW

    pltpu.sync_copy(idx_hbm.at[pl.ds(wid * CPW, CPW)], idx_v)
    pltpu.sync_copy(w_hbm, w_v)

    w0 = w_v[pl.ds(0, 16)]
    w1 = w_v[pl.ds(16, 16)]
    w2 = w_v[pl.ds(32, 16)]

    lane = lax.iota(jnp.int32, 16)

    sems = (sem0, sem1)

    def fire(k, buf):
        return [
            pltpu.async_copy(tab_hbm.at[idx_v.at[k, f]],
                             rows_v.at[buf, f], sems[buf])
            for f in range(F)
        ]

    pending = fire(0, 0)
    for k in range(CPW):
        cur = k % 2
        drain = pending
        if k + 1 < CPW:
            pending = fire(k + 1, 1 - cur)
        for c in drain:
            c.wait()
        pltpu.sync_copy(idxf_hbm.at[wid * CPW + k], idxf_v)

        def group_body(g, carry):
            res = jnp.zeros((16,), jnp.float32)
            for l in range(16):
                b = g * 16 + l
                v = rows_v[cur, 0, b, :]
                acc = v
                acc2 = v * v
                for f in range(1, F):
                    v = rows_v[cur, f, b, :]
                    acc = acc + v
                    acc2 = acc2 + v * v
                t = acc * acc - acc2
                lv0 = idxf_v[b, 0, :]
                lv1 = idxf_v[b, 1, :]
                s = 0.5 * t + lv0 * w0 + lv1 * w1 + w2
                res = jnp.where(lane == l, jnp.sum(s), res)
            out_v[pl.ds(g * 16, 16)] = res
            return carry

        lax.fori_loop(0, CHUNK // 16, group_body, 0)
        pltpu.sync_copy(out_v, out_hbm.at[pl.ds(base + k * CHUNK, CHUNK)])


def kernel(sparse_inputs, tables, W, b):
    flat_tab = tables.reshape(F * V1, D)
    idx = sparse_inputs.astype(jnp.int32) + (
        jnp.arange(F, dtype=jnp.int32) * V1)[None, :]
    idx_arr = idx.reshape(B // CHUNK, CHUNK, F).transpose(0, 2, 1)
    idxf = jnp.zeros((B, 32), jnp.float32).at[:, :F].set(
        sparse_inputs.astype(jnp.float32))
    idxf_arr = idxf.reshape(B // CHUNK, CHUNK, 2, 16)
    w_pad = jnp.zeros((48,), jnp.float32)
    w_pad = w_pad.at[0:F].set(W[0].astype(jnp.float32))
    w_pad = w_pad.at[32].set(b[0].astype(jnp.float32))

    mesh = plsc.VectorSubcoreMesh(core_axis_name="c", subcore_axis_name="s")
    run = pl.kernel(
        _fm_body,
        out_type=jax.ShapeDtypeStruct((B,), jnp.float32),
        mesh=mesh,
        compiler_params=pltpu.CompilerParams(
            needs_layout_passes=False, use_tc_tiling_on_sc=False),
        scratch_types=[
            pltpu.VMEM((CPW, F, CHUNK), jnp.int32),
            pltpu.VMEM((2, F, CHUNK, D), jnp.float32),
            pltpu.VMEM((CHUNK, 2, 16), jnp.float32),
            pltpu.VMEM((CHUNK,), jnp.float32),
            pltpu.VMEM((48,), jnp.float32),
            pltpu.SemaphoreType.DMA,
            pltpu.SemaphoreType.DMA,
        ],
    )
    return run(flat_tab, idx_arr, idxf_arr, w_pad)


# speedup vs baseline: 2.2827x; 2.2827x over previous
import jax
import jax.numpy as jnp
from jax import lax
from jax.experimental import pallas as pl
from jax.experimental.pallas import tpu as pltpu
from jax.experimental.pallas import tpu_sc as plsc

B = 16384
F = 26
VOCAB = 100000
VP = 102400
D = 16
NC = 2
NS = 16
NW = NC * NS
CHUNK = 128
ROWS_PW = B // NW
CPW = ROWS_PW // CHUNK
VB = 512
NVB = (VOCAB + 1 + VB - 1) // VB


def _relayout_body(in_ref, out_ref):
    blk = in_ref[0]
    t = blk.T
    out_ref[...] = jnp.concatenate(
        [t[j * (VB // 8):(j + 1) * (VB // 8), :] for j in range(8)], axis=1)


def _fm_body(tab_hbm, idx_hbm, idxf_hbm, w_hbm, out_hbm,
             idx_v, rows_v, idxf_v, out_v, w_v, sem0, sem1):
    wid = lax.axis_index("s") * NC + lax.axis_index("c")
    base = wid * ROWS_PW

    pltpu.sync_copy(idx_hbm.at[pl.ds(wid * CPW, CPW)], idx_v)
    pltpu.sync_copy(w_hbm, w_v)

    w0 = w_v[pl.ds(0, 16)]
    w1 = w_v[pl.ds(16, 16)]
    w2 = w_v[pl.ds(32, 16)]

    lane = lax.iota(jnp.int32, 16)

    sems = (sem0, sem1)

    def fire(k, buf):
        return [
            pltpu.async_copy(tab_hbm.at[idx_v.at[k, f]],
                             rows_v.at[buf, f], sems[buf])
            for f in range(F)
        ]

    pending = fire(0, 0)
    for k in range(CPW):
        cur = k % 2
        drain = pending
        if k + 1 < CPW:
            pending = fire(k + 1, 1 - cur)
        for c in drain:
            c.wait()
        pltpu.sync_copy(idxf_hbm.at[wid * CPW + k], idxf_v)

        def group_body(g, carry):
            res = jnp.zeros((16,), jnp.float32)
            for l in range(16):
                b = g * 16 + l
                v = rows_v[cur, 0, b, :]
                acc = v
                acc2 = v * v
                for f in range(1, F):
                    v = rows_v[cur, f, b, :]
                    acc = acc + v
                    acc2 = acc2 + v * v
                t = acc * acc - acc2
                lv0 = idxf_v[b, 0, :]
                lv1 = idxf_v[b, 1, :]
                s = 0.5 * t + lv0 * w0 + lv1 * w1 + w2
                res = jnp.where(lane == l, jnp.sum(s), res)
            out_v[pl.ds(g * 16, 16)] = res
            return carry

        lax.fori_loop(0, CHUNK // 16, group_body, 0)
        pltpu.sync_copy(out_v, out_hbm.at[pl.ds(base + k * CHUNK, CHUNK)])


def kernel(sparse_inputs, tables, W, b):
    tab_t = tables.transpose(0, 2, 1)
    flat2 = pl.pallas_call(
        _relayout_body,
        grid=(F, NVB),
        in_specs=[pl.BlockSpec((1, D, VB), lambda f, v: (f, 0, v))],
        out_specs=pl.BlockSpec((VB // 8, 128),
                               lambda f, v: (f * (VP // VB) + v, 0)),
        out_shape=jax.ShapeDtypeStruct((F * VP // 8, 128), jnp.float32),
    )(tab_t)
    flat_tab = flat2.reshape(F * VP, D)

    v = sparse_inputs.astype(jnp.int32)
    v2 = v % 512
    idx = ((v // 512) * 512 + (v2 % 64) * 8 + v2 // 64) + (
        jnp.arange(F, dtype=jnp.int32) * VP)[None, :]
    idx_arr = idx.reshape(B // CHUNK, CHUNK, F).transpose(0, 2, 1)
    idxf = jnp.zeros((B, 32), jnp.float32).at[:, :F].set(
        sparse_inputs.astype(jnp.float32))
    idxf_arr = idxf.reshape(B // CHUNK, CHUNK, 2, 16)
    w_pad = jnp.zeros((48,), jnp.float32)
    w_pad = w_pad.at[0:F].set(W[0].astype(jnp.float32))
    w_pad = w_pad.at[32].set(b[0].astype(jnp.float32))

    mesh = plsc.VectorSubcoreMesh(core_axis_name="c", subcore_axis_name="s")
    run = pl.kernel(
        _fm_body,
        out_type=jax.ShapeDtypeStruct((B,), jnp.float32),
        mesh=mesh,
        compiler_params=pltpu.CompilerParams(
            needs_layout_passes=False, use_tc_tiling_on_sc=False),
        scratch_types=[
            pltpu.VMEM((CPW, F, CHUNK), jnp.int32),
            pltpu.VMEM((2, F, CHUNK, D), jnp.float32),
            pltpu.VMEM((CHUNK, 2, 16), jnp.float32),
            pltpu.VMEM((CHUNK,), jnp.float32),
            pltpu.VMEM((48,), jnp.float32),
            pltpu.SemaphoreType.DMA,
            pltpu.SemaphoreType.DMA,
        ],
    )
    return run(flat_tab, idx_arr, idxf_arr, w_pad)


# speedup vs baseline: 17.2846x; 7.5720x over previous
import jax
import jax.numpy as jnp
from jax import lax
from jax.experimental import pallas as pl
from jax.experimental.pallas import tpu as pltpu
from jax.experimental.pallas import tpu_sc as plsc

B = 16384
F = 26
VOCAB = 100000
VP = 106496
D = 16
NC = 2
NS = 16
NW = NC * NS
CHUNK = 128
ROWS_PW = B // NW
CPW = ROWS_PW // CHUNK
VB = 8192
NVB = (VOCAB + 1 + VB - 1) // VB


def _relayout_body(p_ref, in_ref, out_ref):
    for t in range(VB // 1024):
        sq = in_ref[0, :, t * 1024:(t + 1) * 1024].reshape(128, 128)
        out_ref[t * 128:(t + 1) * 128, :] = lax.dot_general(
            sq, p_ref[...], (((0,), (1,)), ((), ())),
            preferred_element_type=jnp.float32)


def _fm_body(tab_hbm, idx_hbm, idxf_hbm, w_hbm, out_hbm,
             idx_v, rows_v, idxf_v, out_v, w_v, sem0, sem1):
    wid = lax.axis_index("s") * NC + lax.axis_index("c")
    base = wid * ROWS_PW

    pltpu.sync_copy(idx_hbm.at[pl.ds(wid * CPW, CPW)], idx_v)
    pltpu.sync_copy(w_hbm, w_v)

    w0 = w_v[pl.ds(0, 16)]
    w1 = w_v[pl.ds(16, 16)]
    w2 = w_v[pl.ds(32, 16)]

    lane = lax.iota(jnp.int32, 16)

    sems = (sem0, sem1)

    def fire(k, buf):
        return [
            pltpu.async_copy(tab_hbm.at[idx_v.at[k, f]],
                             rows_v.at[buf, f], sems[buf])
            for f in range(F)
        ]

    pending = fire(0, 0)
    for k in range(CPW):
        cur = k % 2
        drain = pending
        if k + 1 < CPW:
            pending = fire(k + 1, 1 - cur)
        for c in drain:
            c.wait()
        pltpu.sync_copy(idxf_hbm.at[wid * CPW + k], idxf_v)

        def group_body(g, carry):
            res = jnp.zeros((16,), jnp.float32)
            for l in range(16):
                b = g * 16 + l
                v = rows_v[cur, 0, b, :]
                acc = v
                acc2 = v * v
                for f in range(1, F):
                    v = rows_v[cur, f, b, :]
                    acc = acc + v
                    acc2 = acc2 + v * v
                t = acc * acc - acc2
                lv0 = idxf_v[b, 0, :]
                lv1 = idxf_v[b, 1, :]
                s = 0.5 * t + lv0 * w0 + lv1 * w1 + w2
                res = jnp.where(lane == l, jnp.sum(s), res)
            out_v[pl.ds(g * 16, 16)] = res
            return carry

        lax.fori_loop(0, CHUNK // 16, group_body, 0)
        pltpu.sync_copy(out_v, out_hbm.at[pl.ds(base + k * CHUNK, CHUNK)])


def kernel(sparse_inputs, tables, W, b):
    tab_t = tables.transpose(0, 2, 1)
    rp = jnp.arange(128, dtype=jnp.int32)
    perm = ((rp % 16) * 8 + rp // 16)
    p_mat = (rp[None, :] == perm[:, None]).astype(jnp.float32)
    flat2 = pl.pallas_call(
        _relayout_body,
        grid=(F, NVB),
        in_specs=[
            pl.BlockSpec((128, 128), lambda f, v: (0, 0)),
            pl.BlockSpec((1, D, VB), lambda f, v: (f, 0, v)),
        ],
        out_specs=pl.BlockSpec((VB // 8, 128),
                               lambda f, v: (f * (VP // VB) + v, 0)),
        out_shape=jax.ShapeDtypeStruct((F * VP // 8, 128), jnp.float32),
    )(p_mat, tab_t)
    flat_tab = flat2.reshape(F * VP, D)

    v = sparse_inputs.astype(jnp.int32)
    idx = ((v // 1024) * 1024 + (v % 128) * 8 + (v % 1024) // 128) + (
        jnp.arange(F, dtype=jnp.int32) * VP)[None, :]
    idx_arr = idx.reshape(B // CHUNK, CHUNK, F).transpose(0, 2, 1)
    idxf = jnp.zeros((B, 32), jnp.float32).at[:, :F].set(
        sparse_inputs.astype(jnp.float32))
    idxf_arr = idxf.reshape(B // CHUNK, CHUNK, 2, 16)
    w_pad = jnp.zeros((48,), jnp.float32)
    w_pad = w_pad.at[0:F].set(W[0].astype(jnp.float32))
    w_pad = w_pad.at[32].set(b[0].astype(jnp.float32))

    mesh = plsc.VectorSubcoreMesh(core_axis_name="c", subcore_axis_name="s")
    run = pl.kernel(
        _fm_body,
        out_type=jax.ShapeDtypeStruct((B,), jnp.float32),
        mesh=mesh,
        compiler_params=pltpu.CompilerParams(
            needs_layout_passes=False, use_tc_tiling_on_sc=False),
        scratch_types=[
            pltpu.VMEM((CPW, F, CHUNK), jnp.int32),
            pltpu.VMEM((2, F, CHUNK, D), jnp.float32),
            pltpu.VMEM((CHUNK, 2, 16), jnp.float32),
            pltpu.VMEM((CHUNK,), jnp.float32),
            pltpu.VMEM((48,), jnp.float32),
            pltpu.SemaphoreType.DMA,
            pltpu.SemaphoreType.DMA,
        ],
    )
    return run(flat_tab, idx_arr, idxf_arr, w_pad)


# speedup vs baseline: 21.3046x; 1.2326x over previous
import jax
import jax.numpy as jnp
from jax import lax
from jax.experimental import pallas as pl
from jax.experimental.pallas import tpu as pltpu
from jax.experimental.pallas import tpu_sc as plsc

B = 16384
F = 26
VOCAB = 100000
VP = 114688
D = 16
NC = 2
NS = 16
NW = NC * NS
CHUNK = 128
ROWS_PW = B // NW
CPW = ROWS_PW // CHUNK
VB = 16384
NVB = (VOCAB + 1 + VB - 1) // VB


def _relayout_body(p_ref, in_ref, out_ref):
    for t in range(VB // 1024):
        sq = in_ref[0, :, t * 1024:(t + 1) * 1024].reshape(128, 128)
        out_ref[t * 128:(t + 1) * 128, :] = lax.dot_general(
            sq, p_ref[...], (((0,), (1,)), ((), ())),
            preferred_element_type=jnp.float32)


def _fm_body(tab_hbm, idx_hbm, idxf_hbm, w_hbm, out_hbm,
             idx_v, rows_v, idxf_v, out_v, w_v, sem0, sem1):
    wid = lax.axis_index("s") * NC + lax.axis_index("c")
    base = wid * ROWS_PW

    pltpu.sync_copy(idx_hbm.at[pl.ds(wid * CPW, CPW)], idx_v)
    pltpu.sync_copy(w_hbm, w_v)

    w0 = w_v[pl.ds(0, 16)]
    w1 = w_v[pl.ds(16, 16)]
    w2 = w_v[pl.ds(32, 16)]

    lane = lax.iota(jnp.int32, 16)

    sems = (sem0, sem1)

    def fire(k, buf):
        return [
            pltpu.async_copy(tab_hbm.at[idx_v.at[k, f]],
                             rows_v.at[buf, f], sems[buf])
            for f in range(F)
        ]

    pending = fire(0, 0)
    for k in range(CPW):
        cur = k % 2
        drain = pending
        if k + 1 < CPW:
            pending = fire(k + 1, 1 - cur)
        for c in drain:
            c.wait()
        pltpu.sync_copy(idxf_hbm.at[wid * CPW + k], idxf_v)

        def group_body(g, carry):
            res = jnp.zeros((16,), jnp.float32)
            for l in range(16):
                b = g * 16 + l
                v = rows_v[cur, 0, b, :]
                acc = v
                acc2 = v * v
                for f in range(1, F):
                    v = rows_v[cur, f, b, :]
                    acc = acc + v
                    acc2 = acc2 + v * v
                t = acc * acc - acc2
                lv0 = idxf_v[b, 0, :]
                lv1 = idxf_v[b, 1, :]
                s = 0.5 * t + lv0 * w0 + lv1 * w1 + w2
                res = jnp.where(lane == l, jnp.sum(s), res)
            out_v[pl.ds(g * 16, 16)] = res
            return carry

        lax.fori_loop(0, CHUNK // 16, group_body, 0)
        pltpu.sync_copy(out_v, out_hbm.at[pl.ds(base + k * CHUNK, CHUNK)])


def kernel(sparse_inputs, tables, W, b):
    tab_t = tables.transpose(0, 2, 1)
    rp = jnp.arange(128, dtype=jnp.int32)
    perm = ((rp % 16) * 8 + rp // 16)
    p_mat = (rp[None, :] == perm[:, None]).astype(jnp.float32)
    flat2 = pl.pallas_call(
        _relayout_body,
        grid=(F, NVB),
        in_specs=[
            pl.BlockSpec((128, 128), lambda f, v: (0, 0)),
            pl.BlockSpec((1, D, VB), lambda f, v: (f, 0, v)),
        ],
        out_specs=pl.BlockSpec((VB // 8, 128),
                               lambda f, v: (f * (VP // VB) + v, 0)),
        out_shape=jax.ShapeDtypeStruct((F * VP // 8, 128), jnp.float32),
    )(p_mat, tab_t)
    flat_tab = flat2.reshape(F * VP, D)

    v = sparse_inputs.astype(jnp.int32)
    idx = ((v // 1024) * 1024 + (v % 128) * 8 + (v % 1024) // 128) + (
        jnp.arange(F, dtype=jnp.int32) * VP)[None, :]
    idx_arr = idx.reshape(B // CHUNK, CHUNK, F).transpose(0, 2, 1)
    idxf = jnp.zeros((B, 32), jnp.float32).at[:, :F].set(
        sparse_inputs.astype(jnp.float32))
    idxf_arr = idxf.reshape(B // CHUNK, CHUNK, 2, 16)
    w_pad = jnp.zeros((48,), jnp.float32)
    w_pad = w_pad.at[0:F].set(W[0].astype(jnp.float32))
    w_pad = w_pad.at[32].set(b[0].astype(jnp.float32))

    mesh = plsc.VectorSubcoreMesh(core_axis_name="c", subcore_axis_name="s")
    run = pl.kernel(
        _fm_body,
        out_type=jax.ShapeDtypeStruct((B,), jnp.float32),
        mesh=mesh,
        compiler_params=pltpu.CompilerParams(
            needs_layout_passes=False, use_tc_tiling_on_sc=False),
        scratch_types=[
            pltpu.VMEM((CPW, F, CHUNK), jnp.int32),
            pltpu.VMEM((2, F, CHUNK, D), jnp.float32),
            pltpu.VMEM((CHUNK, 2, 16), jnp.float32),
            pltpu.VMEM((CHUNK,), jnp.float32),
            pltpu.VMEM((48,), jnp.float32),
            pltpu.SemaphoreType.DMA,
            pltpu.SemaphoreType.DMA,
        ],
    )
    return run(flat_tab, idx_arr, idxf_arr, w_pad)


# speedup vs baseline: 29.6606x; 1.3922x over previous
import jax
import jax.numpy as jnp
from jax import lax
from jax.experimental import pallas as pl
from jax.experimental.pallas import tpu as pltpu
from jax.experimental.pallas import tpu_sc as plsc

B = 16384
F = 26
VOCAB = 100000
VP = 102400
D = 16
NC = 2
NS = 16
NW = NC * NS
CHUNK = 128
ROWS_PW = B // NW
CPW = ROWS_PW // CHUNK
VB = 51200
NVB = (VOCAB + 1 + VB - 1) // VB


def _relayout_body(p_ref, in_ref, out_ref):
    for t in range(VB // 1024):
        sq = in_ref[0, :, t * 1024:(t + 1) * 1024].reshape(128, 128)
        out_ref[t * 128:(t + 1) * 128, :] = lax.dot_general(
            sq, p_ref[...], (((0,), (1,)), ((), ())),
            preferred_element_type=jnp.float32)


def _fm_body(tab_hbm, idx_hbm, idxf_hbm, w_hbm, out_hbm,
             idx_v, rows_v, idxf_v, out_v, w_v, sem0, sem1):
    wid = lax.axis_index("s") * NC + lax.axis_index("c")
    base = wid * ROWS_PW

    pltpu.sync_copy(idx_hbm.at[pl.ds(wid * CPW, CPW)], idx_v)
    pltpu.sync_copy(w_hbm, w_v)

    w0 = w_v[pl.ds(0, 16)]
    w1 = w_v[pl.ds(16, 16)]
    w2 = w_v[pl.ds(32, 16)]

    lane = lax.iota(jnp.int32, 16)

    sems = (sem0, sem1)

    def fire(k, buf):
        return [
            pltpu.async_copy(tab_hbm.at[idx_v.at[k, f]],
                             rows_v.at[buf, f], sems[buf])
            for f in range(F)
        ]

    pending = fire(0, 0)
    for k in range(CPW):
        cur = k % 2
        drain = pending
        if k + 1 < CPW:
            pending = fire(k + 1, 1 - cur)
        for c in drain:
            c.wait()
        pltpu.sync_copy(idxf_hbm.at[wid * CPW + k], idxf_v)

        def group_body(g, carry):
            res = jnp.zeros((16,), jnp.float32)
            for l in range(16):
                b = g * 16 + l
                v = rows_v[cur, 0, b, :]
                acc = v
                acc2 = v * v
                for f in range(1, F):
                    v = rows_v[cur, f, b, :]
                    acc = acc + v
                    acc2 = acc2 + v * v
                t = acc * acc - acc2
                lv0 = idxf_v[b, 0, :]
                lv1 = idxf_v[b, 1, :]
                s = 0.5 * t + lv0 * w0 + lv1 * w1 + w2
                res = jnp.where(lane == l, jnp.sum(s), res)
            out_v[pl.ds(g * 16, 16)] = res
            return carry

        lax.fori_loop(0, CHUNK // 16, group_body, 0)
        pltpu.sync_copy(out_v, out_hbm.at[pl.ds(base + k * CHUNK, CHUNK)])


def kernel(sparse_inputs, tables, W, b):
    tab_t = tables.transpose(0, 2, 1)
    rp = jnp.arange(128, dtype=jnp.int32)
    perm = ((rp % 16) * 8 + rp // 16)
    p_mat = (rp[None, :] == perm[:, None]).astype(jnp.float32)
    flat2 = pl.pallas_call(
        _relayout_body,
        grid=(F, NVB),
        in_specs=[
            pl.BlockSpec((128, 128), lambda f, v: (0, 0)),
            pl.BlockSpec((1, D, VB), lambda f, v: (f, 0, v)),
        ],
        out_specs=pl.BlockSpec((VB // 8, 128),
                               lambda f, v: (f * (VP // VB) + v, 0)),
        out_shape=jax.ShapeDtypeStruct((F * VP // 8, 128), jnp.float32),
    )(p_mat, tab_t)
    flat_tab = flat2.reshape(F * VP, D)

    v = sparse_inputs.astype(jnp.int32)
    idx = ((v // 1024) * 1024 + (v % 128) * 8 + (v % 1024) // 128) + (
        jnp.arange(F, dtype=jnp.int32) * VP)[None, :]
    idx_arr = idx.reshape(B // CHUNK, CHUNK, F).transpose(0, 2, 1)
    idxf = jnp.zeros((B, 32), jnp.float32).at[:, :F].set(
        sparse_inputs.astype(jnp.float32))
    idxf_arr = idxf.reshape(B // CHUNK, CHUNK, 2, 16)
    w_pad = jnp.zeros((48,), jnp.float32)
    w_pad = w_pad.at[0:F].set(W[0].astype(jnp.float32))
    w_pad = w_pad.at[32].set(b[0].astype(jnp.float32))

    mesh = plsc.VectorSubcoreMesh(core_axis_name="c", subcore_axis_name="s")
    run = pl.kernel(
        _fm_body,
        out_type=jax.ShapeDtypeStruct((B,), jnp.float32),
        mesh=mesh,
        compiler_params=pltpu.CompilerParams(
            needs_layout_passes=False, use_tc_tiling_on_sc=False),
        scratch_types=[
            pltpu.VMEM((CPW, F, CHUNK), jnp.int32),
            pltpu.VMEM((2, F, CHUNK, D), jnp.float32),
            pltpu.VMEM((CHUNK, 2, 16), jnp.float32),
            pltpu.VMEM((CHUNK,), jnp.float32),
            pltpu.VMEM((48,), jnp.float32),
            pltpu.SemaphoreType.DMA,
            pltpu.SemaphoreType.DMA,
        ],
    )
    return run(flat_tab, idx_arr, idxf_arr, w_pad)


# speedup vs baseline: 31.3638x; 1.0574x over previous
import jax
import jax.numpy as jnp
from jax import lax
from jax.experimental import pallas as pl
from jax.experimental.pallas import tpu as pltpu
from jax.experimental.pallas import tpu_sc as plsc

B = 16384
F = 26
VOCAB = 100000
VP = 102400
D = 16
NC = 2
NS = 16
NW = NC * NS
CHUNK = 128
ROWS_PW = B // NW
CPW = ROWS_PW // CHUNK
VB = 102400
NVB = (VOCAB + 1 + VB - 1) // VB


def _relayout_body(p_ref, in_ref, out_ref):
    for t in range(VB // 1024):
        sq = in_ref[0, :, t * 1024:(t + 1) * 1024].reshape(128, 128)
        out_ref[t * 128:(t + 1) * 128, :] = lax.dot_general(
            sq, p_ref[...], (((0,), (1,)), ((), ())),
            preferred_element_type=jnp.float32)


def _fm_body(tab_hbm, idx_hbm, idxf_hbm, w_hbm, out_hbm,
             idx_v, rows_v, idxf_v, out_v, w_v, sem0, sem1):
    wid = lax.axis_index("s") * NC + lax.axis_index("c")
    base = wid * ROWS_PW

    pltpu.sync_copy(idx_hbm.at[pl.ds(wid * CPW, CPW)], idx_v)
    pltpu.sync_copy(w_hbm, w_v)

    w0 = w_v[pl.ds(0, 16)]
    w1 = w_v[pl.ds(16, 16)]
    w2 = w_v[pl.ds(32, 16)]

    lane = lax.iota(jnp.int32, 16)

    sems = (sem0, sem1)

    def fire(k, buf):
        return [
            pltpu.async_copy(tab_hbm.at[idx_v.at[k, f]],
                             rows_v.at[buf, f], sems[buf])
            for f in range(F)
        ]

    pending = fire(0, 0)
    for k in range(CPW):
        cur = k % 2
        drain = pending
        if k + 1 < CPW:
            pending = fire(k + 1, 1 - cur)
        for c in drain:
            c.wait()
        pltpu.sync_copy(idxf_hbm.at[wid * CPW + k], idxf_v)

        def group_body(g, carry):
            res = jnp.zeros((16,), jnp.float32)
            for l in range(16):
                b = g * 16 + l
                v = rows_v[cur, 0, b, :]
                acc = v
                acc2 = v * v
                for f in range(1, F):
                    v = rows_v[cur, f, b, :]
                    acc = acc + v
                    acc2 = acc2 + v * v
                t = acc * acc - acc2
                lv0 = idxf_v[b, 0, :]
                lv1 = idxf_v[b, 1, :]
                s = 0.5 * t + lv0 * w0 + lv1 * w1 + w2
                res = jnp.where(lane == l, jnp.sum(s), res)
            out_v[pl.ds(g * 16, 16)] = res
            return carry

        lax.fori_loop(0, CHUNK // 16, group_body, 0)
        pltpu.sync_copy(out_v, out_hbm.at[pl.ds(base + k * CHUNK, CHUNK)])


def kernel(sparse_inputs, tables, W, b):
    tab_t = tables.transpose(0, 2, 1)
    rp = jnp.arange(128, dtype=jnp.int32)
    perm = ((rp % 16) * 8 + rp // 16)
    p_mat = (rp[None, :] == perm[:, None]).astype(jnp.float32)
    flat2 = pl.pallas_call(
        _relayout_body,
        grid=(F, NVB),
        in_specs=[
            pl.BlockSpec((128, 128), lambda f, v: (0, 0)),
            pl.BlockSpec((1, D, VB), lambda f, v: (f, 0, v)),
        ],
        out_specs=pl.BlockSpec((VB // 8, 128),
                               lambda f, v: (f * (VP // VB) + v, 0)),
        out_shape=jax.ShapeDtypeStruct((F * VP // 8, 128), jnp.float32),
    )(p_mat, tab_t)
    flat_tab = flat2.reshape(F * VP, D)

    v = sparse_inputs.astype(jnp.int32)
    idx = ((v // 1024) * 1024 + (v % 128) * 8 + (v % 1024) // 128) + (
        jnp.arange(F, dtype=jnp.int32) * VP)[None, :]
    idx_arr = idx.reshape(B // CHUNK, CHUNK, F).transpose(0, 2, 1)
    idxf = jnp.zeros((B, 32), jnp.float32).at[:, :F].set(
        sparse_inputs.astype(jnp.float32))
    idxf_arr = idxf.reshape(B // CHUNK, CHUNK, 2, 16)
    w_pad = jnp.zeros((48,), jnp.float32)
    w_pad = w_pad.at[0:F].set(W[0].astype(jnp.float32))
    w_pad = w_pad.at[32].set(b[0].astype(jnp.float32))

    mesh = plsc.VectorSubcoreMesh(core_axis_name="c", subcore_axis_name="s")
    run = pl.kernel(
        _fm_body,
        out_type=jax.ShapeDtypeStruct((B,), jnp.float32),
        mesh=mesh,
        compiler_params=pltpu.CompilerParams(
            needs_layout_passes=False, use_tc_tiling_on_sc=False),
        scratch_types=[
            pltpu.VMEM((CPW, F, CHUNK), jnp.int32),
            pltpu.VMEM((2, F, CHUNK, D), jnp.float32),
            pltpu.VMEM((CHUNK, 2, 16), jnp.float32),
            pltpu.VMEM((CHUNK,), jnp.float32),
            pltpu.VMEM((48,), jnp.float32),
            pltpu.SemaphoreType.DMA,
            pltpu.SemaphoreType.DMA,
        ],
    )
    return run(flat_tab, idx_arr, idxf_arr, w_pad)


# speedup vs baseline: 37.0801x; 1.1823x over previous
import jax
import jax.numpy as jnp
from jax import lax
from jax.experimental import pallas as pl
from jax.experimental.pallas import tpu as pltpu
from jax.experimental.pallas import tpu_sc as plsc

B = 16384
F = 26
VOCAB = 100000
VP = 102400
D = 16
NC = 2
NS = 16
NW = NC * NS
CHUNK = 128
ROWS_PW = B // NW
CPW = ROWS_PW // CHUNK
VB = 102400
NVB = (VOCAB + 1 + VB - 1) // VB


def _relayout_body(p_ref, in_ref, out_ref):
    for t in range(VB // 1024):
        sq = in_ref[0, :, t * 1024:(t + 1) * 1024].reshape(128, 128)
        out_ref[t * 128:(t + 1) * 128, :] = lax.dot_general(
            sq, p_ref[...], (((0,), (1,)), ((), ())),
            preferred_element_type=jnp.float32)


def _fm_body(tab_hbm, raw_hbm, w_hbm, out_hbm,
             raw_v, idx_v, rows_v, out_v, w_v, sem0, sem1):
    wid = lax.axis_index("s") * NC + lax.axis_index("c")
    base = wid * ROWS_PW

    pltpu.sync_copy(w_hbm, w_v)

    w0 = w_v[pl.ds(0, 16)]
    w1 = w_v[pl.ds(16, 16)]
    w2 = w_v[pl.ds(32, 16)]

    lane = lax.iota(jnp.int32, 16)
    lane_hi = jnp.minimum(lane + 16, F - 1)

    sems = (sem0, sem1)

    def build(k, s):
        pltpu.sync_copy(raw_hbm.at[pl.ds(base + k * CHUNK, CHUNK)],
                        raw_v.at[s])

        def per_field(f, carry):
            fvec = jnp.full((16,), f, jnp.int32)
            for g in range(CHUNK // 16):
                vals = plsc.load_gather(raw_v.at[s], [lane + g * 16, fvec])
                p = ((vals // 1024) * 1024 + (vals % 128) * 8
                     + (vals % 1024) // 128 + f * VP)
                idx_v[s, f, pl.ds(g * 16, 16)] = p
            return carry

        lax.fori_loop(0, F, per_field, 0)

    def fire(buf):
        return [
            pltpu.async_copy(tab_hbm.at[idx_v.at[buf, f]],
                             rows_v.at[buf, f], sems[buf])
            for f in range(F)
        ]

    build(0, 0)
    pending = fire(0)
    for k in range(CPW):
        cur = k % 2
        drain = pending
        if k + 1 < CPW:
            build(k + 1, 1 - cur)
            pending = fire(1 - cur)
        for c in drain:
            c.wait()

        def group_body(g, carry):
            res = jnp.zeros((16,), jnp.float32)
            for l in range(16):
                b = g * 16 + l
                v = rows_v[cur, 0, b, :]
                acc = v
                acc2 = v * v
                for f in range(1, F):
                    v = rows_v[cur, f, b, :]
                    acc = acc + v
                    acc2 = acc2 + v * v
                t = acc * acc - acc2
                bvec = jnp.full((16,), b, jnp.int32)
                lv0 = plsc.load_gather(raw_v.at[cur], [bvec, lane]
                                       ).astype(jnp.float32)
                lv1 = plsc.load_gather(raw_v.at[cur], [bvec, lane_hi]
                                       ).astype(jnp.float32)
                s = 0.5 * t + lv0 * w0 + lv1 * w1 + w2
                res = jnp.where(lane == l, jnp.sum(s), res)
            out_v[pl.ds(g * 16, 16)] = res
            return carry

        lax.fori_loop(0, CHUNK // 16, group_body, 0)
        pltpu.sync_copy(out_v, out_hbm.at[pl.ds(base + k * CHUNK, CHUNK)])


def kernel(sparse_inputs, tables, W, b):
    tab_t = tables.transpose(0, 2, 1)
    rp = jnp.arange(128, dtype=jnp.int32)
    perm = ((rp % 16) * 8 + rp // 16)
    p_mat = (rp[None, :] == perm[:, None]).astype(jnp.float32)
    flat2 = pl.pallas_call(
        _relayout_body,
        grid=(F, NVB),
        in_specs=[
            pl.BlockSpec((128, 128), lambda f, v: (0, 0)),
            pl.BlockSpec((1, D, VB), lambda f, v: (f, 0, v)),
        ],
        out_specs=pl.BlockSpec((VB // 8, 128),
                               lambda f, v: (f * (VP // VB) + v, 0)),
        out_shape=jax.ShapeDtypeStruct((F * VP // 8, 128), jnp.float32),
    )(p_mat, tab_t)
    flat_tab = flat2.reshape(F * VP, D)

    raw = sparse_inputs.astype(jnp.int32)
    w_pad = jnp.zeros((48,), jnp.float32)
    w_pad = w_pad.at[0:F].set(W[0].astype(jnp.float32))
    w_pad = w_pad.at[32].set(b[0].astype(jnp.float32))

    mesh = plsc.VectorSubcoreMesh(core_axis_name="c", subcore_axis_name="s")
    run = pl.kernel(
        _fm_body,
        out_type=jax.ShapeDtypeStruct((B,), jnp.float32),
        mesh=mesh,
        compiler_params=pltpu.CompilerParams(
            needs_layout_passes=False, use_tc_tiling_on_sc=False),
        scratch_types=[
            pltpu.VMEM((2, CHUNK, F), jnp.int32),
            pltpu.VMEM((2, F, CHUNK), jnp.int32),
            pltpu.VMEM((2, F, CHUNK, D), jnp.float32),
            pltpu.VMEM((CHUNK,), jnp.float32),
            pltpu.VMEM((48,), jnp.float32),
            pltpu.SemaphoreType.DMA,
            pltpu.SemaphoreType.DMA,
        ],
    )
    return run(flat_tab, raw, w_pad)
